# initial kernel scaffold (unmeasured)
import jax
import jax.numpy as jnp
from jax import lax
from jax.experimental import pallas as pl
from jax.experimental.pallas import tpu as pltpu


def kernel(Q, K, V):
    B, Sq, H, D = Q.shape
    Skv = K.shape[1]
    BH = B * H
    scale = D ** -0.5

    def body(q_ref, k_ref, v_ref, out_ref, o_comm, l_comm, send_sems, recv_sems):
        b = pl.program_id(0)
        h = pl.program_id(1)
        i = b * H + h
        x = lax.axis_index("x")
        y = lax.axis_index("y")
        z = lax.axis_index("z")
        peer = (x, y, 1 - z)

        @pl.when(jnp.logical_and(b == 0, h == 0))
        def _entry_barrier():
            bsem = pltpu.get_barrier_semaphore()
            pl.semaphore_signal(
                bsem, inc=1, device_id=peer, device_id_type=pl.DeviceIdType.MESH
            )
            pl.semaphore_wait(bsem, 1)

        q = q_ref[0, :, 0, :].astype(jnp.bfloat16)
        k = k_ref[0, :, 0, :].astype(jnp.bfloat16)
        v = v_ref[0, :, 0, :].astype(jnp.bfloat16)
        s = lax.dot_general(
            q, k, (((1,), (1,)), ((), ())), preferred_element_type=jnp.float32
        )
        p = jnp.exp(s * scale)
        l = jnp.sum(p, axis=-1, keepdims=True)
        o = lax.dot_general(
            p.astype(jnp.bfloat16), v, (((1,), (0,)), ((), ())),
            preferred_element_type=jnp.float32,
        )
        o_comm[0, pl.ds(i, 1)] = o.astype(jnp.bfloat16)[None]
        l_comm[0, pl.ds(i, 1)] = l[None]

        @pl.when(jnp.logical_and(b == B - 1, h == H - 1))
        def _exchange_and_combine():
            rdma_o = pltpu.make_async_remote_copy(
                src_ref=o_comm.at[0],
                dst_ref=o_comm.at[1],
                send_sem=send_sems.at[0],
                recv_sem=recv_sems.at[0],
                device_id=peer,
                device_id_type=pl.DeviceIdType.MESH,
            )
            rdma_l = pltpu.make_async_remote_copy(
                src_ref=l_comm.at[0],
                dst_ref=l_comm.at[1],
                send_sem=send_sems.at[1],
                recv_sem=recv_sems.at[1],
                device_id=peer,
                device_id_type=pl.DeviceIdType.MESH,
            )
            rdma_o.start()
            rdma_l.start()
            rdma_o.wait()
            rdma_l.wait()
            for j in range(BH):
                o_tot = o_comm[0, j].astype(jnp.float32) + o_comm[1, j].astype(
                    jnp.float32
                )
                l_tot = l_comm[0, j] + l_comm[1, j]
                out_ref[j] = o_tot / l_tot

    out = pl.pallas_call(
        body,
        grid=(B, H),
        in_specs=[
            pl.BlockSpec((1, Sq, 1, D), lambda b, h: (b, 0, h, 0)),
            pl.BlockSpec((1, Skv, 1, D), lambda b, h: (b, 0, h, 0)),
            pl.BlockSpec((1, Skv, 1, D), lambda b, h: (b, 0, h, 0)),
        ],
        out_specs=pl.BlockSpec((BH, Sq, D), lambda b, h: (0, 0, 0)),
        out_shape=jax.ShapeDtypeStruct((BH, Sq, D), jnp.float32),
        scratch_shapes=[
            pltpu.VMEM((2, BH, Sq, D), jnp.bfloat16),
            pltpu.VMEM((2, BH, Sq, 1), jnp.float32),
            pltpu.SemaphoreType.DMA((2,)),
            pltpu.SemaphoreType.DMA((2,)),
        ],
        compiler_params=pltpu.CompilerParams(
            dimension_semantics=("arbitrary", "arbitrary"),
            collective_id=0,
        ),
    )(Q, K, V)
    return out.reshape(B, H, Sq, D).transpose(0, 2, 1, 3)


# baseline (device time: 205886 ns/iter reference)
import jax
import jax.numpy as jnp
from jax import lax
from jax.experimental import pallas as pl
from jax.experimental.pallas import tpu as pltpu


def kernel(Q, K, V):
    B, Sq, H, D = Q.shape
    Skv = K.shape[1]
    BH = B * H
    scale = D ** -0.5

    def body(q_ref, k_ref, v_ref, out_ref, o_comm, l_comm, send_sems, recv_sems):
        b = pl.program_id(0)
        h = pl.program_id(1)
        i = b * H + h
        x = lax.axis_index("x")
        y = lax.axis_index("y")
        z = lax.axis_index("z")
        peer = (x, y, 1 - z)

        @pl.when(jnp.logical_and(b == 0, h == 0))
        def _entry_barrier():
            bsem = pltpu.get_barrier_semaphore()
            pl.semaphore_signal(
                bsem, inc=1, device_id=peer, device_id_type=pl.DeviceIdType.MESH
            )
            pl.semaphore_wait(bsem, 1)

        q = q_ref[0].astype(jnp.bfloat16)
        k = k_ref[0].astype(jnp.bfloat16)
        v = v_ref[0].astype(jnp.bfloat16)
        s = lax.dot_general(
            q, k, (((1,), (1,)), ((), ())), preferred_element_type=jnp.float32
        )
        p = jnp.exp(s * scale)
        l = jnp.sum(p, axis=-1, keepdims=True)
        o = lax.dot_general(
            p.astype(jnp.bfloat16), v, (((1,), (0,)), ((), ())),
            preferred_element_type=jnp.float32,
        )
        o_comm[0, pl.ds(i, 1)] = o.astype(jnp.bfloat16)[None]
        l_comm[0, pl.ds(i, 1)] = l[None]

        @pl.when(jnp.logical_and(b == B - 1, h == H - 1))
        def _exchange_and_combine():
            rdma_o = pltpu.make_async_remote_copy(
                src_ref=o_comm.at[0],
                dst_ref=o_comm.at[1],
                send_sem=send_sems.at[0],
                recv_sem=recv_sems.at[0],
                device_id=peer,
                device_id_type=pl.DeviceIdType.MESH,
            )
            rdma_l = pltpu.make_async_remote_copy(
                src_ref=l_comm.at[0],
                dst_ref=l_comm.at[1],
                send_sem=send_sems.at[1],
                recv_sem=recv_sems.at[1],
                device_id=peer,
                device_id_type=pl.DeviceIdType.MESH,
            )
            rdma_o.start()
            rdma_l.start()
            rdma_o.wait()
            rdma_l.wait()
            for j in range(BH):
                o_tot = o_comm[0, j].astype(jnp.float32) + o_comm[1, j].astype(
                    jnp.float32
                )
                l_tot = l_comm[0, j] + l_comm[1, j]
                out_ref[j] = o_tot / l_tot

    out = pl.pallas_call(
        body,
        grid=(B, H),
        in_specs=[
            pl.BlockSpec((1, Sq, D), lambda b, h: (b, 0, h)),
            pl.BlockSpec((1, Skv, D), lambda b, h: (b, 0, h)),
            pl.BlockSpec((1, Skv, D), lambda b, h: (b, 0, h)),
        ],
        out_specs=pl.BlockSpec((BH, Sq, D), lambda b, h: (0, 0, 0)),
        out_shape=jax.ShapeDtypeStruct((BH, Sq, D), jnp.float32),
        scratch_shapes=[
            pltpu.VMEM((2, BH, Sq, D), jnp.bfloat16),
            pltpu.VMEM((2, BH, Sq, 1), jnp.float32),
            pltpu.SemaphoreType.DMA((2,)),
            pltpu.SemaphoreType.DMA((2,)),
        ],
        compiler_params=pltpu.CompilerParams(
            dimension_semantics=("arbitrary", "arbitrary"),
            collective_id=0,
        ),
    )(Q.reshape(B, Sq, H * D), K.reshape(B, Skv, H * D), V.reshape(B, Skv, H * D))
    return out.reshape(B, H, Sq, D).transpose(0, 2, 1, 3)
